# baseline (speedup 1.0000x reference)
import functools

import jax
import jax.numpy as jnp
from jax import lax
from jax.experimental import pallas as pl
from jax.experimental.pallas import tpu as pltpu
from jax.experimental.pallas import tpu_sc as plsc

E = 16
F = 26
D = F * E
V = 100000
B = 4096
NC, NS = 2, 16
NW = NC * NS
SPW = B // NW
SPC = 4
NCHUNK = SPW // SPC
RPC = SPC * F

_MESH = plsc.VectorSubcoreMesh(
    core_axis_name="c", subcore_axis_name="s", num_cores=NC, num_subcores=NS)


def _ffm_body(emb_hbm, idx_hbm, val_hbm, w1_hbm, out_hbm,
              idx_v, val_v, rows_v, w1_v, res_v, sem_r, sem_w):
    wid = lax.axis_index("s") * NC + lax.axis_index("c")
    sbase = wid * SPW
    rbase = sbase * F
    pltpu.sync_copy(idx_hbm.at[pl.ds(rbase, SPW * F)], idx_v)
    pltpu.sync_copy(val_hbm.at[pl.ds(rbase, SPW * F)],
                    val_v.at[pl.ds(0, SPW * F)])
    lane = lax.iota(jnp.int32, E)
    tail_mask = lane < (F - E)

    def chunk_body(ck, resvec):
        idx_slice = idx_v.at[pl.ds(ck * RPC, RPC)]
        cp_r = pltpu.async_copy(emb_hbm.at[idx_slice], rows_v, sem_r)
        cp_w = pltpu.async_copy(w1_hbm.at[idx_slice],
                                w1_v.at[pl.ds(0, RPC)], sem_w)
        cp_r.wait()
        cp_w.wait()

        def samp_body(sl, rv):
            voff = ck * RPC + sl * F
            v0 = val_v[pl.ds(voff, E)]
            v1 = val_v[pl.ds(voff + E, E)]
            xs = [v0[i] for i in range(E)] + [v1[i] for i in range(F - E)]
            r0 = sl * F
            u0 = w1_v[pl.ds(r0, E)]
            u1 = w1_v[pl.ds(r0 + E, E)]
            fo = jnp.sum(v0 * u0) + jnp.sum(
                jnp.where(tail_mask, v1 * u1, jnp.float32(0.0)))
            acc = jnp.zeros((E,), jnp.float32)
            for i in range(F):
                for j in range(i + 1, F):
                    vi = rows_v[r0 + i, j, :]
                    vj = rows_v[r0 + j, i, :]
                    acc = acc + (xs[i] * xs[j]) * (vi * vj)
            total = jnp.sum(acc) + fo
            gs = ck * SPC + sl
            return jnp.where(lane == gs % E, total, rv)

        resvec = lax.fori_loop(0, SPC, samp_body, resvec)

        @pl.when((ck % (E // SPC)) == (E // SPC - 1))
        def _():
            res_v[pl.ds((ck // (E // SPC)) * E, E)] = resvec

        return resvec

    lax.fori_loop(0, NCHUNK, chunk_body, jnp.zeros((E,), jnp.float32))
    pltpu.sync_copy(res_v, out_hbm.at[pl.ds(sbase, SPW)])


@jax.jit
def _ffm_call(emb2d, idx_flat, val_flat, w1_flat):
    run = pl.kernel(
        _ffm_body,
        out_type=jax.ShapeDtypeStruct((B,), jnp.float32),
        mesh=_MESH,
        compiler_params=pltpu.CompilerParams(
            needs_layout_passes=False, use_tc_tiling_on_sc=False),
        scratch_types=[
            pltpu.VMEM((SPW * F,), jnp.int32),
            pltpu.VMEM((SPW * F + E,), jnp.float32),
            pltpu.VMEM((RPC, F, E), jnp.float32),
            pltpu.VMEM((RPC + 2 * E,), jnp.float32),
            pltpu.VMEM((SPW,), jnp.float32),
            pltpu.SemaphoreType.DMA,
            pltpu.SemaphoreType.DMA,
        ],
    )
    return run(emb2d, idx_flat, val_flat, w1_flat)


def kernel(feature_idx, feature_values, feature_embeddings,
           weights_first_order, fm_bias):
    idx_flat = feature_idx.reshape(-1).astype(jnp.int32)
    val_flat = feature_values.reshape(-1)
    out = _ffm_call(feature_embeddings, idx_flat, val_flat,
                    weights_first_order.reshape(-1))
    return out.reshape(B, 1) + fm_bias


# speedup vs baseline: 3.3544x; 3.3544x over previous
import jax
import jax.numpy as jnp
from jax import lax
from jax.experimental import pallas as pl
from jax.experimental.pallas import tpu as pltpu
from jax.experimental.pallas import tpu_sc as plsc

E = 16
F = 26
D = F * E
DP = 512
V = 100000
B = 4096
NC, NS = 2, 16
NW = NC * NS
SPW = B // NW
SPC = 4
NCHUNK = SPW // SPC
RPC = SPC * F

VC = 512
NVB = -(-V // VC)

_MESH = plsc.VectorSubcoreMesh(
    core_axis_name="c", subcore_axis_name="s", num_cores=NC, num_subcores=NS)


def _relayout_body(in_ref, out_ref):
    x = in_ref[...]
    x2 = x.reshape(D, VC)
    xp = jnp.concatenate(
        [x2, jnp.zeros((DP - D, VC), jnp.float32)], axis=0)
    y = jnp.transpose(xp, (1, 0))
    out_ref[...] = y.reshape(VC * DP // 128, 128)


def _relayout(emb_t):
    return pl.pallas_call(
        _relayout_body,
        grid=(NVB,),
        in_specs=[pl.BlockSpec((F, E, VC), lambda i: (0, 0, i))],
        out_specs=pl.BlockSpec((VC * DP // 128, 128), lambda i: (i, 0)),
        out_shape=jax.ShapeDtypeStruct((V * DP // 128, 128), jnp.float32),
    )(emb_t)


def _ffm_body(emb_hbm, idx_hbm, val_hbm, w1_hbm, out_hbm,
              idx_v, val_v, rows_v, w1_v, res_v, sem_r, sem_w):
    wid = lax.axis_index("s") * NC + lax.axis_index("c")
    sbase = wid * SPW
    rbase = sbase * F
    pltpu.sync_copy(idx_hbm.at[pl.ds(rbase, SPW * F)], idx_v)
    pltpu.sync_copy(val_hbm.at[pl.ds(rbase, SPW * F)],
                    val_v.at[pl.ds(0, SPW * F)])
    lane = lax.iota(jnp.int32, E)
    tail_mask = lane < (F - E)

    def chunk_body(ck, resvec):
        idx_slice = idx_v.at[pl.ds(ck * RPC, RPC)]
        cp_r = pltpu.async_copy(emb_hbm.at[idx_slice], rows_v, sem_r)
        cp_w = pltpu.async_copy(w1_hbm.at[idx_slice],
                                w1_v.at[pl.ds(0, RPC)], sem_w)
        cp_r.wait()
        cp_w.wait()

        def samp_body(sl, rv):
            voff = ck * RPC + sl * F
            v0 = val_v[pl.ds(voff, E)]
            v1 = val_v[pl.ds(voff + E, E)]
            xs = [v0[i] for i in range(E)] + [v1[i] for i in range(F - E)]
            r0 = sl * F
            u0 = w1_v[pl.ds(r0, E)]
            u1 = w1_v[pl.ds(r0 + E, E)]
            fo = jnp.sum(v0 * u0) + jnp.sum(
                jnp.where(tail_mask, v1 * u1, jnp.float32(0.0)))
            acc = jnp.zeros((E,), jnp.float32)
            for i in range(F):
                for j in range(i + 1, F):
                    vi = rows_v[r0 + i, pl.ds(j * E, E)]
                    vj = rows_v[r0 + j, pl.ds(i * E, E)]
                    acc = acc + (xs[i] * xs[j]) * (vi * vj)
            total = jnp.sum(acc) + fo
            gs = ck * SPC + sl
            return jnp.where(lane == gs % E, total, rv)

        resvec = lax.fori_loop(0, SPC, samp_body, resvec)

        @pl.when((ck % (E // SPC)) == (E // SPC - 1))
        def _():
            res_v[pl.ds((ck // (E // SPC)) * E, E)] = resvec

        return resvec

    lax.fori_loop(0, NCHUNK, chunk_body, jnp.zeros((E,), jnp.float32))
    pltpu.sync_copy(res_v, out_hbm.at[pl.ds(sbase, SPW)])


@jax.jit
def _ffm_call(table_pad, idx_flat, val_flat, w1_flat):
    run = pl.kernel(
        _ffm_body,
        out_type=jax.ShapeDtypeStruct((B,), jnp.float32),
        mesh=_MESH,
        compiler_params=pltpu.CompilerParams(
            needs_layout_passes=False, use_tc_tiling_on_sc=False),
        scratch_types=[
            pltpu.VMEM((SPW * F,), jnp.int32),
            pltpu.VMEM((SPW * F + E,), jnp.float32),
            pltpu.VMEM((RPC, DP), jnp.float32),
            pltpu.VMEM((RPC + 2 * E,), jnp.float32),
            pltpu.VMEM((SPW,), jnp.float32),
            pltpu.SemaphoreType.DMA,
            pltpu.SemaphoreType.DMA,
        ],
    )
    return run(table_pad, idx_flat, val_flat, w1_flat)


def kernel(feature_idx, feature_values, feature_embeddings,
           weights_first_order, fm_bias):
    idx_flat = feature_idx.reshape(-1).astype(jnp.int32)
    val_flat = feature_values.reshape(-1)
    emb_t = jnp.transpose(feature_embeddings, (1, 2, 0))
    table_pad = _relayout(emb_t).reshape(V, DP)
    out = _ffm_call(table_pad, idx_flat, val_flat,
                    weights_first_order.reshape(-1))
    return out.reshape(B, 1) + fm_bias


# speedup vs baseline: 5.2485x; 1.5647x over previous
import jax
import jax.numpy as jnp
from jax import lax
from jax.experimental import pallas as pl
from jax.experimental.pallas import tpu as pltpu
from jax.experimental.pallas import tpu_sc as plsc

E = 16
F = 26
D = F * E
DP = 512
V = 100000
B = 4096
NC, NS = 2, 16
NW = NC * NS
SPW = B // NW
SPC = 4
NCHUNK = SPW // SPC
RPC = SPC * F

VC = 2048
NVB = -(-V // VC)

_MESH = plsc.VectorSubcoreMesh(
    core_axis_name="c", subcore_axis_name="s", num_cores=NC, num_subcores=NS)


def _relayout_body(in_ref, out_ref):
    x = in_ref[...]
    x2 = x.reshape(D, VC)
    xp = jnp.concatenate(
        [x2, jnp.zeros((DP - D, VC), jnp.float32)], axis=0)
    y = jnp.transpose(xp, (1, 0))
    out_ref[...] = y.reshape(VC * DP // 128, 128)


def _relayout(emb_t):
    return pl.pallas_call(
        _relayout_body,
        grid=(NVB,),
        in_specs=[pl.BlockSpec((F, E, VC), lambda i: (0, 0, i))],
        out_specs=pl.BlockSpec((VC * DP // 128, 128), lambda i: (i, 0)),
        out_shape=jax.ShapeDtypeStruct((V * DP // 128, 128), jnp.float32),
    )(emb_t)


def _ffm_body(emb_hbm, idx_hbm, val_hbm, w1_hbm, out_hbm,
              idx_v, val_v, rows_v, w1_v, res_v,
              sem_r0, sem_r1, sem_w0, sem_w1):
    wid = lax.axis_index("s") * NC + lax.axis_index("c")
    sbase = wid * SPW
    rbase = sbase * F
    pltpu.sync_copy(idx_hbm.at[pl.ds(rbase, SPW * F)], idx_v)
    pltpu.sync_copy(val_hbm.at[pl.ds(rbase, SPW * F)],
                    val_v.at[pl.ds(0, SPW * F)])
    lane = lax.iota(jnp.int32, E)
    tail_mask = lane < (F - E)
    sems_r = (sem_r0, sem_r1)
    sems_w = (sem_w0, sem_w1)

    def copies(ck, par):
        isl = idx_v.at[pl.ds(ck * RPC, RPC)]
        return (pltpu.make_async_copy(emb_hbm.at[isl],
                                      rows_v.at[par], sems_r[par]),
                pltpu.make_async_copy(w1_hbm.at[isl],
                                      w1_v.at[par, pl.ds(0, RPC)],
                                      sems_w[par]))

    for c in copies(0, 0):
        c.start()

    def pair_body(m, resvec):
        for par in (0, 1):
            ck = 2 * m + par

            @pl.when(ck + 1 < NCHUNK)
            def _():
                for c in copies(ck + 1, 1 - par):
                    c.start()

            for c in copies(ck, par):
                c.wait()

            def samp_body(sl, rv):
                voff = ck * RPC + sl * F
                v0 = val_v[pl.ds(voff, E)]
                v1 = val_v[pl.ds(voff + E, E)]
                xs = ([v0[i] for i in range(E)]
                      + [v1[i] for i in range(F - E)])
                r0 = sl * F
                u0 = w1_v[par, pl.ds(r0, E)]
                u1 = w1_v[par, pl.ds(r0 + E, E)]
                fo = jnp.sum(v0 * u0) + jnp.sum(
                    jnp.where(tail_mask, v1 * u1, jnp.float32(0.0)))
                acc = jnp.zeros((E,), jnp.float32)
                for i in range(F):
                    for j in range(i + 1, F):
                        vi = rows_v[par, r0 + i, pl.ds(j * E, E)]
                        vj = rows_v[par, r0 + j, pl.ds(i * E, E)]
                        acc = acc + (xs[i] * xs[j]) * (vi * vj)
                total = jnp.sum(acc) + fo
                gs = ck * SPC + sl
                return jnp.where(lane == gs % E, total, rv)

            resvec = lax.fori_loop(0, SPC, samp_body, resvec)

            @pl.when((ck % (E // SPC)) == (E // SPC - 1))
            def _():
                res_v[pl.ds((ck // (E // SPC)) * E, E)] = resvec

        return resvec

    lax.fori_loop(0, NCHUNK // 2, pair_body, jnp.zeros((E,), jnp.float32))
    pltpu.sync_copy(res_v, out_hbm.at[pl.ds(sbase, SPW)])


@jax.jit
def _ffm_call(table_pad, idx_flat, val_flat, w1_flat):
    run = pl.kernel(
        _ffm_body,
        out_type=jax.ShapeDtypeStruct((B,), jnp.float32),
        mesh=_MESH,
        compiler_params=pltpu.CompilerParams(
            needs_layout_passes=False, use_tc_tiling_on_sc=False),
        scratch_types=[
            pltpu.VMEM((SPW * F,), jnp.int32),
            pltpu.VMEM((SPW * F + E,), jnp.float32),
            pltpu.VMEM((2, RPC, DP), jnp.float32),
            pltpu.VMEM((2, RPC + 2 * E,), jnp.float32),
            pltpu.VMEM((SPW,), jnp.float32),
            pltpu.SemaphoreType.DMA,
            pltpu.SemaphoreType.DMA,
            pltpu.SemaphoreType.DMA,
            pltpu.SemaphoreType.DMA,
        ],
    )
    return run(table_pad, idx_flat, val_flat, w1_flat)


def kernel(feature_idx, feature_values, feature_embeddings,
           weights_first_order, fm_bias):
    idx_flat = feature_idx.reshape(-1).astype(jnp.int32)
    val_flat = feature_values.reshape(-1)
    emb_t = jnp.transpose(feature_embeddings, (1, 2, 0))
    table_pad = _relayout(emb_t).reshape(V, DP)
    out = _ffm_call(table_pad, idx_flat, val_flat,
                    weights_first_order.reshape(-1))
    return out.reshape(B, 1) + fm_bias
